# baseline (speedup 1.0000x reference)
import jax
import jax.numpy as jnp
from jax import lax
from jax.experimental import pallas as pl
from jax.experimental.pallas import tpu as pltpu

_K = 32
_NEG = -1e30


def _body(aw_row_ref, aw_col_ref, f_ref, edW_ref, edb_ref, euW_ref, eub_ref,
          ndW_ref, ndb_ref, nuW_ref, nub_ref, o_ref):
    F = f_ref[0]
    N = F.shape[0]

    sq = jnp.sum(F * F, axis=1, keepdims=True)
    inv = 1.0 / jnp.maximum(jnp.sqrt(sq), 1e-12)
    Fn = F * inv
    sim = lax.dot_general(Fn, Fn, (((1,), (1,)), ((), ())),
                          preferred_element_type=jnp.float32)

    row = lax.broadcasted_iota(jnp.int32, (N, N), 0)
    col = lax.broadcasted_iota(jnp.int32, (N, N), 1)
    aw_r = aw_row_ref[0]
    sim = jnp.where((row == col) | (col == 0), _NEG, sim)
    sim = jnp.where(row == 0, aw_r, sim)

    bits = lax.bitcast_convert_type(sim, jnp.int32)
    ki = jnp.where(bits < 0, bits ^ jnp.int32(0x7FFFFFFF), bits)
    ones8 = jnp.ones((N, 8), jnp.float32)
    SGN = jnp.int32(-2147483648)
    blocks = [(0, 144), (144, 144), (288, 144), (432, 145)]
    kb = [lax.slice(ki, (r0, 0), (r0 + rs, N)) for r0, rs in blocks]
    kcount = jnp.float32(_K)
    u = [jnp.zeros((rs, 1), jnp.int32) for _, rs in blocks]
    for j in range(31, -1, -1):
        bit = jnp.int32(-2147483648 if j == 31 else (1 << j))
        for r in range(4):
            cand = u[r] | bit
            maskf = jnp.where(kb[r] >= (cand ^ SGN), 1.0, 0.0)
            cnt = jnp.dot(maskf, ones8,
                          preferred_element_type=jnp.float32)[:, 0:1]
            u[r] = jnp.where(cnt >= kcount, cand, u[r])
    uf = jnp.concatenate(u, axis=0) ^ SGN
    t = lax.bitcast_convert_type(
        jnp.where(uf < 0, uf ^ jnp.int32(0x7FFFFFFF), uf), jnp.float32)

    e = jnp.where(sim >= t, jnp.exp(sim - 1.0), 0.0)
    z = jnp.sum(e, axis=1, keepdims=True)
    A = e / z
    A = jnp.where(row == col, 1.0, A)

    awc = aw_col_ref[0]
    t0 = t[0, 0]
    e0 = jnp.where(awc >= t0, jnp.exp(awc - 1.0), 0.0)
    r_col = e0 / jnp.sum(e0)
    rowc = lax.broadcasted_iota(jnp.int32, (N, 1), 0)
    a0c = jnp.where(rowc == 0, 1.0, r_col)
    A = jnp.where(col == 0, a0c, A)

    bf = jnp.bfloat16
    Ab = A.astype(bf)
    HE = jnp.dot(Ab, F.astype(bf), preferred_element_type=jnp.float32)
    h = jnp.dot(HE.astype(bf), edW_ref[...], preferred_element_type=jnp.float32) + edb_ref[...]
    h = jnp.where(h >= 0, h, 0.2 * h)
    HEr = jnp.dot(h.astype(bf), euW_ref[...], preferred_element_type=jnp.float32) + eub_ref[...]
    HC = lax.dot_general(Ab, HEr.astype(bf), (((0,), (0,)), ((), ())),
                         preferred_element_type=jnp.float32)
    h2 = jnp.dot(HC.astype(bf), ndW_ref[...], preferred_element_type=jnp.float32) + ndb_ref[...]
    h2 = jnp.where(h2 >= 0, h2, 0.2 * h2)
    out = jnp.dot(h2.astype(bf), nuW_ref[...], preferred_element_type=jnp.float32) + nub_ref[...]
    o_ref[0] = out


def _build_call(B, N, D, H, interpret=False):
    return pl.pallas_call(
        _body,
        grid=(B,),
        in_specs=[
            pl.BlockSpec((1, 1, N), lambda b: (b, 0, 0)),
            pl.BlockSpec((1, N, 1), lambda b: (b, 0, 0)),
            pl.BlockSpec((1, N, D), lambda b: (b, 0, 0)),
            pl.BlockSpec((D, H), lambda b: (0, 0)),
            pl.BlockSpec((1, H), lambda b: (0, 0)),
            pl.BlockSpec((H, D), lambda b: (0, 0)),
            pl.BlockSpec((1, D), lambda b: (0, 0)),
            pl.BlockSpec((D, H), lambda b: (0, 0)),
            pl.BlockSpec((1, H), lambda b: (0, 0)),
            pl.BlockSpec((H, D), lambda b: (0, 0)),
            pl.BlockSpec((1, D), lambda b: (0, 0)),
        ],
        out_specs=pl.BlockSpec((1, N, D), lambda b: (b, 0, 0)),
        out_shape=jax.ShapeDtypeStruct((B, N, D), jnp.float32),
        compiler_params=pltpu.CompilerParams(
            dimension_semantics=("arbitrary",)),
        interpret=interpret,
    )


def kernel(features, attn_weights, edge_down_W, edge_down_b, edge_up_W,
           edge_up_b, node_down_W, node_down_b, node_up_W, node_up_b):
    B, N, D = features.shape
    H = edge_down_W.shape[1]
    aw_pad = jnp.concatenate(
        [jnp.full((B, 1), _NEG, features.dtype), attn_weights], axis=1)
    call = _build_call(B, N, D, H)
    bf = jnp.bfloat16
    return call(aw_pad[:, None, :], aw_pad[:, :, None], features,
                edge_down_W.astype(bf), edge_down_b.reshape(1, -1),
                edge_up_W.astype(bf), edge_up_b.reshape(1, -1),
                node_down_W.astype(bf), node_down_b.reshape(1, -1),
                node_up_W.astype(bf), node_up_b.reshape(1, -1))


# speedup vs baseline: 1.3857x; 1.3857x over previous
import jax
import jax.numpy as jnp
from jax import lax
from jax.experimental import pallas as pl
from jax.experimental.pallas import tpu as pltpu

_K = 32
_NEG = -1e30


def _body(aw_row_ref, aw_col_ref, f_ref, edW_ref, edb_ref, euW_ref, eub_ref,
          ndW_ref, ndb_ref, nuW_ref, nub_ref, o_ref):
    F = f_ref[0]
    N = F.shape[0]

    sq = jnp.sum(F * F, axis=1, keepdims=True)
    inv = 1.0 / jnp.maximum(jnp.sqrt(sq), 1e-12)
    Fn = F * inv
    sim = lax.dot_general(Fn, Fn, (((1,), (1,)), ((), ())),
                          preferred_element_type=jnp.float32)

    row = lax.broadcasted_iota(jnp.int32, (N, N), 0)
    col = lax.broadcasted_iota(jnp.int32, (N, N), 1)
    aw_r = aw_row_ref[0]
    sim = jnp.where((row == col) | (col == 0), _NEG, sim)
    sim = jnp.where(row == 0, aw_r, sim)

    t = jnp.max(sim, axis=1, keepdims=True)
    m1 = t
    for _ in range(_K - 1):
        t = jnp.max(jnp.where(sim >= t, _NEG, sim), axis=1, keepdims=True)

    e = jnp.where(sim >= t, jnp.exp(sim - m1), 0.0)
    z = jnp.sum(e, axis=1, keepdims=True)
    A = e / z
    A = jnp.where(row == col, 1.0, A)

    awc = aw_col_ref[0]
    t0 = t[0, 0]
    m0 = m1[0, 0]
    e0 = jnp.where(awc >= t0, jnp.exp(awc - m0), 0.0)
    r_col = e0 / jnp.sum(e0)
    rowc = lax.broadcasted_iota(jnp.int32, (N, 1), 0)
    a0c = jnp.where(rowc == 0, 1.0, r_col)
    A = jnp.where(col == 0, a0c, A)

    bf = jnp.bfloat16
    Ab = A.astype(bf)
    HE = jnp.dot(Ab, F.astype(bf), preferred_element_type=jnp.float32)
    h = jnp.dot(HE.astype(bf), edW_ref[...], preferred_element_type=jnp.float32) + edb_ref[...]
    h = jnp.where(h >= 0, h, 0.2 * h)
    HEr = jnp.dot(h.astype(bf), euW_ref[...], preferred_element_type=jnp.float32) + eub_ref[...]
    HC = lax.dot_general(Ab, HEr.astype(bf), (((0,), (0,)), ((), ())),
                         preferred_element_type=jnp.float32)
    h2 = jnp.dot(HC.astype(bf), ndW_ref[...], preferred_element_type=jnp.float32) + ndb_ref[...]
    h2 = jnp.where(h2 >= 0, h2, 0.2 * h2)
    out = jnp.dot(h2.astype(bf), nuW_ref[...], preferred_element_type=jnp.float32) + nub_ref[...]
    o_ref[0] = out


def _build_call(B, N, D, H, interpret=False):
    return pl.pallas_call(
        _body,
        grid=(B,),
        in_specs=[
            pl.BlockSpec((1, 1, N), lambda b: (b, 0, 0)),
            pl.BlockSpec((1, N, 1), lambda b: (b, 0, 0)),
            pl.BlockSpec((1, N, D), lambda b: (b, 0, 0)),
            pl.BlockSpec((D, H), lambda b: (0, 0)),
            pl.BlockSpec((1, H), lambda b: (0, 0)),
            pl.BlockSpec((H, D), lambda b: (0, 0)),
            pl.BlockSpec((1, D), lambda b: (0, 0)),
            pl.BlockSpec((D, H), lambda b: (0, 0)),
            pl.BlockSpec((1, H), lambda b: (0, 0)),
            pl.BlockSpec((H, D), lambda b: (0, 0)),
            pl.BlockSpec((1, D), lambda b: (0, 0)),
        ],
        out_specs=pl.BlockSpec((1, N, D), lambda b: (b, 0, 0)),
        out_shape=jax.ShapeDtypeStruct((B, N, D), jnp.float32),
        compiler_params=pltpu.CompilerParams(
            dimension_semantics=("arbitrary",)),
        interpret=interpret,
    )


def kernel(features, attn_weights, edge_down_W, edge_down_b, edge_up_W,
           edge_up_b, node_down_W, node_down_b, node_up_W, node_up_b):
    B, N, D = features.shape
    H = edge_down_W.shape[1]
    aw_pad = jnp.concatenate(
        [jnp.full((B, 1), _NEG, features.dtype), attn_weights], axis=1)
    call = _build_call(B, N, D, H)
    bf = jnp.bfloat16
    return call(aw_pad[:, None, :], aw_pad[:, :, None], features,
                edge_down_W.astype(bf), edge_down_b.reshape(1, -1),
                edge_up_W.astype(bf), edge_up_b.reshape(1, -1),
                node_down_W.astype(bf), node_down_b.reshape(1, -1),
                node_up_W.astype(bf), node_up_b.reshape(1, -1))
